# Initial kernel scaffold; baseline (speedup 1.0000x reference)
import jax
import jax.numpy as jnp
from jax.experimental import pallas as pl


def kernel(x, x_lens, W_t1, b_t1, W_t2, b_t2, embed, Wi_p, Wh_p, b_p, W_jf, W_jg, b_j, W_jo, b_jo):
    raise NotImplementedError("write your pallas kernel here")


# speedup vs baseline: 22.7901x; 22.7901x over previous
import jax
import jax.numpy as jnp
from jax.experimental import pallas as pl
from jax.experimental.pallas import tpu as pltpu

T, N, C = 128, 32, 240
TRANS_H = 512
PRED_H = 256
PRED_L = 2
JOINT_H = 512
VOCAB = 29
BLANK = 28
SOS = 28
MAX_SYM = 2
MAX_ITERS = T * (MAX_SYM + 1) + 8
MAX_OUT = T * MAX_SYM
F_CHUNK = 512


def _decode_kernel(x_ref, xlens_ref, wt1_ref, bt1_ref, wt2_ref, bt2_ref,
                   emb_ref, wl0_ref, bl0_ref, wl1_ref, bl1_ref,
                   wj_ref, bj_ref, wjo_ref, bjo_ref,
                   res_ref, reslen_ref, scores_ref, f_ref):
    def trans_step(i, _):
        xc = x_ref[pl.ds(i * F_CHUNK, F_CHUNK), :]
        h1 = jnp.tanh(jnp.dot(xc, wt1_ref[:, :],
                              preferred_element_type=jnp.float32) + bt1_ref[:, :])
        f_ref[pl.ds(i * F_CHUNK, F_CHUNK), :] = jnp.tanh(
            jnp.dot(h1, wt2_ref[:, :], preferred_element_type=jnp.float32)
            + bt2_ref[:, :])
        return 0
    jax.lax.fori_loop(0, (T * N) // F_CHUNK, trans_step, 0)

    f_lens = xlens_ref[:, :]
    pred_g = jnp.full((N, 1), SOS, jnp.int32)
    h0 = jnp.zeros((N, PRED_H), jnp.float32)
    c0 = jnp.zeros((N, PRED_H), jnp.float32)
    h1 = jnp.zeros((N, PRED_H), jnp.float32)
    c1 = jnp.zeros((N, PRED_H), jnp.float32)
    symbols_added = jnp.zeros((N, 1), jnp.int32)
    time_idx = jnp.zeros((N, 1), jnp.int32)
    finish = (f_lens == 0).astype(jnp.int32)
    res = jnp.zeros((N, MAX_OUT), jnp.int32)
    res_lens = jnp.zeros((N, 1), jnp.int32)
    scores = jnp.zeros((N, 1), jnp.float32)
    fi = f_ref[0:N, :]
    it = jnp.int32(0)

    vocab_iota = jax.lax.broadcasted_iota(jnp.int32, (N, VOCAB), 1)
    col_iota = jax.lax.broadcasted_iota(jnp.int32, (N, MAX_OUT), 1)

    def lstm_cell(xin, h, c, w_ref, b_ref):
        both = jnp.concatenate([xin, h], axis=1)
        gates = jnp.dot(both, w_ref[:, :],
                        preferred_element_type=jnp.float32) + b_ref[:, :]
        ig = jax.nn.sigmoid(gates[:, 0 * PRED_H:1 * PRED_H])
        fg = jax.nn.sigmoid(gates[:, 1 * PRED_H:2 * PRED_H])
        gg = jnp.tanh(gates[:, 2 * PRED_H:3 * PRED_H])
        og = jax.nn.sigmoid(gates[:, 3 * PRED_H:4 * PRED_H])
        c_new = fg * c + ig * gg
        h_new = og * jnp.tanh(c_new)
        return h_new, c_new

    def body_fun(carry):
        (it, pred_g, h0, c0, h1, c1, symbols_added, finish_i, time_idx,
         res, res_lens, scores, fi) = carry
        finish = finish_i != 0
        it = it + 1
        oh = (pred_g == vocab_iota).astype(jnp.float32)
        xl = jnp.dot(oh, emb_ref[0:VOCAB, :],
                     preferred_element_type=jnp.float32)
        nh0, nc0 = lstm_cell(xl, h0, c0, wl0_ref, bl0_ref)
        nh1, nc1 = lstm_cell(nh0, h1, c1, wl1_ref, bl1_ref)
        g_out = nh1
        fg_cat = jnp.concatenate([fi, g_out], axis=1)
        hid = jax.nn.relu(jnp.dot(fg_cat, wj_ref[:, :],
                                  preferred_element_type=jnp.float32)
                          + bj_ref[:, :])
        y = jnp.dot(hid, wjo_ref[:, :],
                    preferred_element_type=jnp.float32) + bjo_ref[:, :]
        ymax = jnp.max(y, axis=1, keepdims=True)
        symbols = jnp.min(jnp.where(y == ymax, vocab_iota, VOCAB),
                          axis=1, keepdims=True)
        update_g = (symbols != BLANK) & (symbols_added != MAX_SYM) & (~finish)
        res = res + jnp.where(update_g & (col_iota == res_lens), symbols, 0)
        scores = scores + jnp.where(update_g, ymax, 0.0)
        ug_i = update_g.astype(jnp.int32)
        res_lens = res_lens + ug_i
        symbols_added = symbols_added + ug_i
        pred_g = jnp.where(update_g, symbols, pred_g)
        h0 = jnp.where(update_g, nh0, h0)
        c0 = jnp.where(update_g, nc0, c0)
        h1 = jnp.where(update_g, nh1, h1)
        c1 = jnp.where(update_g, nc1, c1)
        update_f = (~update_g) & (~finish)
        time_idx = time_idx + update_f.astype(jnp.int32)
        finish_i = ((finish_i != 0) | (time_idx >= f_lens)).astype(jnp.int32)
        t_clamped = jnp.clip(jnp.minimum(time_idx, f_lens - 1), 0, T - 1)
        rows = [f_ref[pl.ds(t_clamped[n, 0] * N + n, 1), :] for n in range(N)]
        fi = jnp.concatenate(rows, axis=0)
        symbols_added = jnp.where(update_f, 0, symbols_added)
        return (it, pred_g, h0, c0, h1, c1, symbols_added, finish_i, time_idx,
                res, res_lens, scores, fi)

    carry = (it, pred_g, h0, c0, h1, c1, symbols_added, finish, time_idx,
             res, res_lens, scores, fi)

    def cond2(carry):
        return jnp.logical_and(carry[0] < MAX_ITERS,
                               jnp.logical_not(jnp.all(carry[7] != 0)))

    carry = jax.lax.while_loop(cond2, body_fun, carry)
    res_ref[:, :] = carry[9]
    reslen_ref[:, :] = carry[10]
    scores_ref[:, :] = carry[11]


def kernel(x, x_lens, W_t1, b_t1, W_t2, b_t2, embed, Wi_p, Wh_p, b_p,
           W_jf, W_jg, b_j, W_jo, b_jo):
    x2d = x.reshape(T * N, C)
    xlens2 = x_lens.reshape(N, 1)
    wl0 = jnp.concatenate([Wi_p[0], Wh_p[0]], axis=0)
    wl1 = jnp.concatenate([Wi_p[1], Wh_p[1]], axis=0)
    bl0 = b_p[0:1, :]
    bl1 = b_p[1:2, :]
    wj = jnp.concatenate([W_jf, W_jg], axis=0)
    res, res_lens, scores = pl.pallas_call(
        _decode_kernel,
        out_shape=[
            jax.ShapeDtypeStruct((N, MAX_OUT), jnp.int32),
            jax.ShapeDtypeStruct((N, 1), jnp.int32),
            jax.ShapeDtypeStruct((N, 1), jnp.float32),
        ],
        scratch_shapes=[pltpu.VMEM((T * N, TRANS_H), jnp.float32)],
    )(x2d, xlens2, W_t1, b_t1.reshape(1, TRANS_H), W_t2,
      b_t2.reshape(1, TRANS_H), embed, wl0, bl0, wl1, bl1,
      wj, b_j.reshape(1, JOINT_H), W_jo, b_jo.reshape(1, VOCAB))
    return res, res_lens.reshape(N), scores.reshape(N)
